# baseline (speedup 1.0000x reference)
import functools

import jax
import jax.numpy as jnp
from jax import lax
from jax.experimental import pallas as pl
from jax.experimental.pallas import tpu as pltpu
from jax.experimental.pallas import tpu_sc as plsc

NUM_CONCEPTS = 128
H1 = 512
H2 = 256
BATCH = 16384

NC = 2
NS = 16
NW = NC * NS
CHUNK = 64
NSPLIT = 2
PB = BATCH // NSPLIT
BT = 2048


def _sc_gather_combine(pb, row0, uid2, qid2, user_table, qdiff_table,
                       qdisc_table, Q_table):
  mesh = plsc.VectorSubcoreMesh(core_axis_name="c", subcore_axis_name="s")
  b_per_w = pb // NW
  nchunk = b_per_w // CHUNK

  @functools.partial(
      pl.kernel,
      mesh=mesh,
      out_type=(
          jax.ShapeDtypeStruct((pb, NUM_CONCEPTS), jnp.float32),
          jax.ShapeDtypeStruct((pb,), jnp.float32),
      ),
      scratch_types=(
          pltpu.VMEM((nchunk, CHUNK), jnp.int32),
          pltpu.VMEM((nchunk, CHUNK), jnp.int32),
          pltpu.VMEM((2, CHUNK, NUM_CONCEPTS), jnp.float32),
          pltpu.VMEM((2, CHUNK, NUM_CONCEPTS), jnp.float32),
          pltpu.VMEM((2, CHUNK, NUM_CONCEPTS), jnp.float32),
          pltpu.VMEM((2, CHUNK), jnp.float32),
          (pltpu.SemaphoreType.DMA, pltpu.SemaphoreType.DMA),
          pltpu.SemaphoreType.DMA,
      ),
  )
  def k(uid_hbm, qid_hbm, ut_hbm, qd_hbm, qs_hbm, qm_hbm,
        x_out, s_out,
        uid_v, qid_v, ubuf, dbuf, qbuf, sbuf, gsems, wsem):
    wid = lax.axis_index("s") * NC + lax.axis_index("c")
    pltpu.sync_copy(uid_hbm.at[pl.ds(row0 + wid * nchunk, nchunk)], uid_v)
    pltpu.sync_copy(qid_hbm.at[pl.ds(row0 + wid * nchunk, nchunk)], qid_v)

    def fire(j, p):
      return (
          pltpu.async_copy(ut_hbm.at[uid_v.at[j]], ubuf.at[p], gsems[p]),
          pltpu.async_copy(qd_hbm.at[qid_v.at[j]], dbuf.at[p], gsems[p]),
          pltpu.async_copy(qm_hbm.at[qid_v.at[j]], qbuf.at[p], gsems[p]),
          pltpu.async_copy(qs_hbm.at[qid_v.at[j]], sbuf.at[p], gsems[p]),
      )

    pending = {0: fire(0, 0)}
    writes = []
    for j in range(nchunk):
      p = j % 2
      if j + 1 < nchunk:
        if j >= 1:
          writes.pop(0).wait()
          writes.pop(0).wait()
        pending[j + 1] = fire(j + 1, 1 - p)
      for cp in pending.pop(j):
        cp.wait()

      def body(r, carry, p=p):
        for ci in range(NUM_CONCEPTS // 16):
          sl = pl.ds(ci * 16, 16)
          a = jnp.exp(-ubuf[p, r, sl])
          b = jnp.exp(-dbuf[p, r, sl])
          q = qbuf[p, r, sl]
          ubuf[p, r, sl] = (b - a) / ((1.0 + a) * (1.0 + b)) * q
        return carry

      lax.fori_loop(0, CHUNK, body, 0)
      base = wid * b_per_w + j * CHUNK
      writes.append(pltpu.async_copy(ubuf.at[p], x_out.at[pl.ds(base, CHUNK)],
                                     wsem))
      writes.append(pltpu.async_copy(sbuf.at[p], s_out.at[pl.ds(base, CHUNK)],
                                     wsem))
    for w in writes:
      w.wait()

  return k(uid2, qid2, user_table, qdiff_table, qdisc_table.reshape(-1),
           Q_table)


def _sig(x):
  return 0.5 * jnp.tanh(0.5 * x) + 0.5


def _tc_mlp_body(x_ref, s_ref, w1_ref, b1_ref, w2_ref, b2_ref,
                 w3_ref, b3_ref, out_ref):
  disc = _sig(s_ref[...].reshape(BT, 1)) * 10.0
  x = disc * x_ref[...]
  h = _sig(
      jnp.dot(x, w1_ref[...], preferred_element_type=jnp.float32) + b1_ref[...])
  h = _sig(
      jnp.dot(h, w2_ref[...], preferred_element_type=jnp.float32) + b2_ref[...])
  o = _sig(
      jnp.dot(h, w3_ref[...], preferred_element_type=jnp.float32) + b3_ref[...])
  out_ref[...] = o.reshape(BT)


def _tc_mlp(pb, x, s, W1, b1, W2, b2, W3, b3):
  grid = (pb // BT,)
  return pl.pallas_call(
      _tc_mlp_body,
      grid=grid,
      in_specs=[
          pl.BlockSpec((BT, NUM_CONCEPTS), lambda i: (i, 0)),
          pl.BlockSpec((1, 1, BT), lambda i: (i, 0, 0)),
          pl.BlockSpec((NUM_CONCEPTS, H1), lambda i: (0, 0)),
          pl.BlockSpec((1, H1), lambda i: (0, 0)),
          pl.BlockSpec((H1, H2), lambda i: (0, 0)),
          pl.BlockSpec((1, H2), lambda i: (0, 0)),
          pl.BlockSpec((H2, 1), lambda i: (0, 0)),
          pl.BlockSpec((1, 1), lambda i: (0, 0)),
      ],
      out_specs=pl.BlockSpec((BT,), lambda i: (i,)),
      out_shape=jax.ShapeDtypeStruct((pb,), jnp.float32),
  )(x, s.reshape(pb // BT, 1, BT), W1, b1, W2, b2, W3, b3)


def kernel(user_id, question_id, user_table, qdiff_table, qdisc_table, Q_table,
           W1, b1, W2, b2, W3, b3):
  uid2 = user_id.astype(jnp.int32).reshape(BATCH // CHUNK, CHUNK)
  qid2 = question_id.astype(jnp.int32).reshape(BATCH // CHUNK, CHUNK)
  b1r, b2r, b3r = b1.reshape(1, H1), b2.reshape(1, H2), b3.reshape(1, 1)
  rows = PB // CHUNK
  outs = []
  for part in range(NSPLIT):
    x, s = _sc_gather_combine(PB, part * rows, uid2, qid2,
                              user_table, qdiff_table, qdisc_table, Q_table)
    outs.append(_tc_mlp(PB, x, s, W1, b1r, W2, b2r, W3, b3r))
  return jnp.concatenate(outs)


# speedup vs baseline: 1.0296x; 1.0296x over previous
import functools

import jax
import jax.numpy as jnp
from jax import lax
from jax.experimental import pallas as pl
from jax.experimental.pallas import tpu as pltpu
from jax.experimental.pallas import tpu_sc as plsc

NUM_CONCEPTS = 128
H1 = 512
H2 = 256
BATCH = 16384

NC = 2
NS = 16
NW = NC * NS
CHUNK = 128
NSPLIT = 2
PB = BATCH // NSPLIT
BT = 4096


def _sc_gather_combine(pb, row0, uid2, qid2, user_table, qdiff_table,
                       qdisc_table, Q_table):
  mesh = plsc.VectorSubcoreMesh(core_axis_name="c", subcore_axis_name="s")
  b_per_w = pb // NW
  nchunk = b_per_w // CHUNK

  @functools.partial(
      pl.kernel,
      mesh=mesh,
      out_type=(
          jax.ShapeDtypeStruct((pb, NUM_CONCEPTS), jnp.float32),
          jax.ShapeDtypeStruct((pb,), jnp.float32),
      ),
      scratch_types=(
          pltpu.VMEM((nchunk, CHUNK), jnp.int32),
          pltpu.VMEM((nchunk, CHUNK), jnp.int32),
          pltpu.VMEM((2, CHUNK, NUM_CONCEPTS), jnp.float32),
          pltpu.VMEM((2, CHUNK, NUM_CONCEPTS), jnp.float32),
          pltpu.VMEM((2, CHUNK, NUM_CONCEPTS), jnp.float32),
          pltpu.VMEM((2, CHUNK), jnp.float32),
          (pltpu.SemaphoreType.DMA, pltpu.SemaphoreType.DMA),
          pltpu.SemaphoreType.DMA,
      ),
  )
  def k(uid_hbm, qid_hbm, ut_hbm, qd_hbm, qs_hbm, qm_hbm,
        x_out, s_out,
        uid_v, qid_v, ubuf, dbuf, qbuf, sbuf, gsems, wsem):
    wid = lax.axis_index("s") * NC + lax.axis_index("c")
    pltpu.sync_copy(uid_hbm.at[pl.ds(row0 + wid * nchunk, nchunk)], uid_v)
    pltpu.sync_copy(qid_hbm.at[pl.ds(row0 + wid * nchunk, nchunk)], qid_v)

    def fire(j, p):
      return (
          pltpu.async_copy(ut_hbm.at[uid_v.at[j]], ubuf.at[p], gsems[p]),
          pltpu.async_copy(qd_hbm.at[qid_v.at[j]], dbuf.at[p], gsems[p]),
          pltpu.async_copy(qm_hbm.at[qid_v.at[j]], qbuf.at[p], gsems[p]),
          pltpu.async_copy(qs_hbm.at[qid_v.at[j]], sbuf.at[p], gsems[p]),
      )

    pending = {0: fire(0, 0)}
    writes = []
    for j in range(nchunk):
      p = j % 2
      if j + 1 < nchunk:
        if j >= 1:
          writes.pop(0).wait()
          writes.pop(0).wait()
        pending[j + 1] = fire(j + 1, 1 - p)
      for cp in pending.pop(j):
        cp.wait()

      def body(r, carry, p=p):
        for ci in range(NUM_CONCEPTS // 16):
          sl = pl.ds(ci * 16, 16)
          a = jnp.exp(-ubuf[p, r, sl])
          b = jnp.exp(-dbuf[p, r, sl])
          q = qbuf[p, r, sl]
          ubuf[p, r, sl] = (b - a) / ((1.0 + a) * (1.0 + b)) * q
        return carry

      lax.fori_loop(0, CHUNK, body, 0)
      base = wid * b_per_w + j * CHUNK
      writes.append(pltpu.async_copy(ubuf.at[p], x_out.at[pl.ds(base, CHUNK)],
                                     wsem))
      writes.append(pltpu.async_copy(sbuf.at[p], s_out.at[pl.ds(base, CHUNK)],
                                     wsem))
    for w in writes:
      w.wait()

  return k(uid2, qid2, user_table, qdiff_table, qdisc_table.reshape(-1),
           Q_table)


def _sig(x):
  return 0.5 * jnp.tanh(0.5 * x) + 0.5


def _tc_mlp_body(x_ref, s_ref, w1_ref, b1_ref, w2_ref, b2_ref,
                 w3_ref, b3_ref, out_ref):
  disc = _sig(s_ref[...].reshape(BT, 1)) * 10.0
  x = disc * x_ref[...]
  h = _sig(
      jnp.dot(x, w1_ref[...], preferred_element_type=jnp.float32) + b1_ref[...])
  h = _sig(
      jnp.dot(h, w2_ref[...], preferred_element_type=jnp.float32) + b2_ref[...])
  o = _sig(
      jnp.dot(h, w3_ref[...], preferred_element_type=jnp.float32) + b3_ref[...])
  out_ref[...] = o.reshape(BT)


def _tc_mlp(pb, x, s, W1, b1, W2, b2, W3, b3):
  grid = (pb // BT,)
  return pl.pallas_call(
      _tc_mlp_body,
      grid=grid,
      in_specs=[
          pl.BlockSpec((BT, NUM_CONCEPTS), lambda i: (i, 0)),
          pl.BlockSpec((1, 1, BT), lambda i: (i, 0, 0)),
          pl.BlockSpec((NUM_CONCEPTS, H1), lambda i: (0, 0)),
          pl.BlockSpec((1, H1), lambda i: (0, 0)),
          pl.BlockSpec((H1, H2), lambda i: (0, 0)),
          pl.BlockSpec((1, H2), lambda i: (0, 0)),
          pl.BlockSpec((H2, 1), lambda i: (0, 0)),
          pl.BlockSpec((1, 1), lambda i: (0, 0)),
      ],
      out_specs=pl.BlockSpec((BT,), lambda i: (i,)),
      out_shape=jax.ShapeDtypeStruct((pb,), jnp.float32),
  )(x, s.reshape(pb // BT, 1, BT), W1, b1, W2, b2, W3, b3)


def kernel(user_id, question_id, user_table, qdiff_table, qdisc_table, Q_table,
           W1, b1, W2, b2, W3, b3):
  uid2 = user_id.astype(jnp.int32).reshape(BATCH // CHUNK, CHUNK)
  qid2 = question_id.astype(jnp.int32).reshape(BATCH // CHUNK, CHUNK)
  b1r, b2r, b3r = b1.reshape(1, H1), b2.reshape(1, H2), b3.reshape(1, 1)
  rows = PB // CHUNK
  outs = []
  for part in range(NSPLIT):
    x, s = _sc_gather_combine(PB, part * rows, uid2, qid2,
                              user_table, qdiff_table, qdisc_table, Q_table)
    outs.append(_tc_mlp(PB, x, s, W1, b1r, W2, b2r, W3, b3r))
  return jnp.concatenate(outs)


# speedup vs baseline: 1.0530x; 1.0228x over previous
import functools

import jax
import jax.numpy as jnp
from jax import lax
from jax.experimental import pallas as pl
from jax.experimental.pallas import tpu as pltpu
from jax.experimental.pallas import tpu_sc as plsc

NUM_CONCEPTS = 128
H1 = 512
H2 = 256
BATCH = 16384

NC = 2
NS = 16
NW = NC * NS
CHUNK = 128
NSPLIT = 2
PB = BATCH // NSPLIT
BT = 4096


def _sc_gather_combine(pb, row0, uid2, qid2, user_table, qdiff_table,
                       qdisc_table, Q_table):
  mesh = plsc.VectorSubcoreMesh(core_axis_name="c", subcore_axis_name="s")
  b_per_w = pb // NW
  nchunk = b_per_w // CHUNK

  @functools.partial(
      pl.kernel,
      mesh=mesh,
      out_type=(
          jax.ShapeDtypeStruct((pb, NUM_CONCEPTS), jnp.float32),
          jax.ShapeDtypeStruct((pb,), jnp.float32),
      ),
      scratch_types=(
          pltpu.VMEM((nchunk, CHUNK), jnp.int32),
          pltpu.VMEM((nchunk, CHUNK), jnp.int32),
          pltpu.VMEM((2, CHUNK, NUM_CONCEPTS), jnp.float32),
          pltpu.VMEM((2, CHUNK, NUM_CONCEPTS), jnp.float32),
          pltpu.VMEM((2, CHUNK, NUM_CONCEPTS), jnp.float32),
          pltpu.VMEM((2, CHUNK), jnp.float32),
          (pltpu.SemaphoreType.DMA, pltpu.SemaphoreType.DMA),
          pltpu.SemaphoreType.DMA,
      ),
  )
  def k(uid_hbm, qid_hbm, ut_hbm, qd_hbm, qs_hbm, qm_hbm,
        x_out, s_out,
        uid_v, qid_v, ubuf, dbuf, qbuf, sbuf, gsems, wsem):
    wid = lax.axis_index("s") * NC + lax.axis_index("c")
    pltpu.sync_copy(uid_hbm.at[pl.ds(row0 + wid * nchunk, nchunk)], uid_v)
    pltpu.sync_copy(qid_hbm.at[pl.ds(row0 + wid * nchunk, nchunk)], qid_v)

    def fire(j, p):
      return (
          pltpu.async_copy(ut_hbm.at[uid_v.at[j]], ubuf.at[p], gsems[p]),
          pltpu.async_copy(qd_hbm.at[qid_v.at[j]], dbuf.at[p], gsems[p]),
          pltpu.async_copy(qm_hbm.at[qid_v.at[j]], qbuf.at[p], gsems[p]),
          pltpu.async_copy(qs_hbm.at[qid_v.at[j]], sbuf.at[p], gsems[p]),
      )

    pending = {0: fire(0, 0)}
    writes = []
    for j in range(nchunk):
      p = j % 2
      if j + 1 < nchunk:
        if j >= 1:
          writes.pop(0).wait()
          writes.pop(0).wait()
        pending[j + 1] = fire(j + 1, 1 - p)
      for cp in pending.pop(j):
        cp.wait()

      def body(r, carry, p=p):
        for ci in range(NUM_CONCEPTS // 16):
          sl = pl.ds(ci * 16, 16)
          a = jnp.exp(-ubuf[p, r, sl])
          b = jnp.exp(-dbuf[p, r, sl])
          q = qbuf[p, r, sl]
          ubuf[p, r, sl] = (b - a) / ((1.0 + a) * (1.0 + b)) * q
        return carry

      lax.fori_loop(0, CHUNK, body, 0)
      base = wid * b_per_w + j * CHUNK
      writes.append(pltpu.async_copy(ubuf.at[p], x_out.at[pl.ds(base, CHUNK)],
                                     wsem))
      writes.append(pltpu.async_copy(sbuf.at[p], s_out.at[pl.ds(base, CHUNK)],
                                     wsem))
    for w in writes:
      w.wait()

  return k(uid2, qid2, user_table, qdiff_table, qdisc_table.reshape(-1),
           Q_table)


def _tc_mlp_body(x_ref, s_ref, w1_ref, b1_ref, w2_ref, b2_ref,
                 w3_ref, b3_ref, out_ref):
  disc = 5.0 * jnp.tanh(0.5 * s_ref[...].reshape(BT, 1)) + 5.0
  x = disc * x_ref[...]
  t = jnp.tanh(
      jnp.dot(x, w1_ref[...], preferred_element_type=jnp.float32) + b1_ref[...])
  t = jnp.tanh(
      jnp.dot(t, w2_ref[...], preferred_element_type=jnp.float32) + b2_ref[...])
  o = jnp.tanh(
      jnp.dot(t, w3_ref[...], preferred_element_type=jnp.float32) + b3_ref[...])
  out_ref[...] = 0.5 * o.reshape(BT) + 0.5


def _tc_mlp(pb, x, s, W1, b1, W2, b2, W3, b3):
  grid = (pb // BT,)
  return pl.pallas_call(
      _tc_mlp_body,
      grid=grid,
      in_specs=[
          pl.BlockSpec((BT, NUM_CONCEPTS), lambda i: (i, 0)),
          pl.BlockSpec((1, 1, BT), lambda i: (i, 0, 0)),
          pl.BlockSpec((NUM_CONCEPTS, H1), lambda i: (0, 0)),
          pl.BlockSpec((1, H1), lambda i: (0, 0)),
          pl.BlockSpec((H1, H2), lambda i: (0, 0)),
          pl.BlockSpec((1, H2), lambda i: (0, 0)),
          pl.BlockSpec((H2, 1), lambda i: (0, 0)),
          pl.BlockSpec((1, 1), lambda i: (0, 0)),
      ],
      out_specs=pl.BlockSpec((BT,), lambda i: (i,)),
      out_shape=jax.ShapeDtypeStruct((pb,), jnp.float32),
  )(x, s.reshape(pb // BT, 1, BT), W1, b1, W2, b2, W3, b3)


def kernel(user_id, question_id, user_table, qdiff_table, qdisc_table, Q_table,
           W1, b1, W2, b2, W3, b3):
  uid2 = user_id.astype(jnp.int32).reshape(BATCH // CHUNK, CHUNK)
  qid2 = question_id.astype(jnp.int32).reshape(BATCH // CHUNK, CHUNK)
  w1f = 0.5 * W1
  b1f = (0.5 * b1).reshape(1, H1)
  w2f = 0.25 * W2
  b2f = (0.5 * b2 + 0.25 * jnp.sum(W2, axis=0)).reshape(1, H2)
  w3f = 0.25 * W3
  b3f = (0.5 * b3 + 0.25 * jnp.sum(W3, axis=0)).reshape(1, 1)
  rows = PB // CHUNK
  outs = []
  for part in range(NSPLIT):
    x, s = _sc_gather_combine(PB, part * rows, uid2, qid2,
                              user_table, qdiff_table, qdisc_table, Q_table)
    outs.append(_tc_mlp(PB, x, s, w1f, b1f, w2f, b2f, w3f, b3f))
  return jnp.concatenate(outs)
